# baseline (speedup 1.0000x reference)
import jax
import jax.numpy as jnp
from jax.experimental import pallas as pl
from jax.experimental.pallas import tpu as pltpu

_K = 99074
_ITERS = 31
_CHUNK = 128


def _haar_bits_kernel(x_ref, bits_ref, exr_ref, exc_ref):
    v = x_ref[0, 0]
    rp = jax.lax.broadcasted_iota(jnp.int32, (512, 512), 0)
    cp = jax.lax.broadcasted_iota(jnp.int32, (512, 512), 1)
    sign = jnp.where(((rp ^ cp) & 1) == 0, 0.5, -0.5)
    z = v * sign
    t = z + jnp.concatenate([z[:, -1:], z[:, :-1]], axis=1)
    u = t + jnp.concatenate([t[-1:, :], t[:-1, :]], axis=0)
    ue = u.reshape(256, 2, 512)[:, 0, :]
    b = jax.lax.bitcast_convert_type(jnp.abs(ue), jnp.int32)
    cpe = jax.lax.broadcasted_iota(jnp.int32, (256, 512), 1)
    bm = jnp.where((cpe & 1) == 1, -1, b)
    bits_ref[0, 0] = bm
    row0 = bm[0:1, :]
    corner = jnp.broadcast_to(bm[0:1, 0:1], (1, 512))
    exr_ref[0, 0] = jnp.where(cpe[0:1, :] == 1, corner, row0)
    exc_ref[0, 0] = bm[:, 0:1]


def _select_kernel(bits_ref, exr_ref, exc_ref, sp_ref, out_ref):
    exr = exr_ref[...]
    exc = exc_ref[...]

    def count(mid):
        def chunk_body(c, acc):
            blk = bits_ref[:, pl.ds(c * _CHUNK, _CHUNK), :]
            return acc + jnp.sum((blk >= mid).astype(jnp.int32),
                                 axis=(1, 2), keepdims=True)
        acc0 = jnp.sum((exr >= mid).astype(jnp.int32), axis=(1, 2),
                       keepdims=True)
        acc0 = acc0 + jnp.sum((exc >= mid).astype(jnp.int32), axis=(1, 2),
                              keepdims=True)
        return jax.lax.fori_loop(0, 768 // _CHUNK, chunk_body, acc0)

    def body(_, carry):
        lo, hi = carry
        mid = lo + (hi - lo + 1) // 2
        ge = count(mid) >= _K
        return jnp.where(ge, mid, lo), jnp.where(ge, hi, mid - 1)

    lo0 = jnp.zeros((16, 1, 1), jnp.int32)
    hi0 = jnp.full((16, 1, 1), 0x7FFFFFFE, jnp.int32)
    lo, _ = jax.lax.fori_loop(0, _ITERS, body, (lo0, hi0))
    med = jax.lax.bitcast_convert_type(lo, jnp.float32)
    std = med / 0.6745
    sp_a = sp_ref[0, 0]
    sp_b = sp_ref[0, 1]
    beta = 1.0 / (sp_a * std + sp_b) ** 2
    out_ref[...] = jnp.broadcast_to(beta.reshape(16, 1), (16, 128))


def kernel(x, a_k, b_k):
    x = x.astype(jnp.float32)

    bits, exr, exc = pl.pallas_call(
        _haar_bits_kernel,
        grid=(16, 3),
        in_specs=[pl.BlockSpec((1, 1, 512, 512), lambda i, j: (i, j, 0, 0))],
        out_specs=[
            pl.BlockSpec((1, 1, 256, 512), lambda i, j: (i, j, 0, 0)),
            pl.BlockSpec((1, 1, 1, 512), lambda i, j: (i, j, 0, 0)),
            pl.BlockSpec((1, 1, 256, 1), lambda i, j: (i, j, 0, 0)),
        ],
        out_shape=[
            jax.ShapeDtypeStruct((16, 3, 256, 512), jnp.int32),
            jax.ShapeDtypeStruct((16, 3, 1, 512), jnp.int32),
            jax.ShapeDtypeStruct((16, 3, 256, 1), jnp.int32),
        ],
    )(x)

    sp = jax.nn.softplus(jnp.stack([a_k, b_k])).reshape(1, 2)
    out = pl.pallas_call(
        _select_kernel,
        in_specs=[
            pl.BlockSpec(memory_space=pltpu.VMEM),
            pl.BlockSpec(memory_space=pltpu.VMEM),
            pl.BlockSpec(memory_space=pltpu.VMEM),
            pl.BlockSpec(memory_space=pltpu.SMEM),
        ],
        out_shape=jax.ShapeDtypeStruct((16, 128), jnp.float32),
    )(bits.reshape(16, 768, 512), exr.reshape(16, 3, 512),
      exc.reshape(16, 768, 1), sp)
    return out[:, 0]
